# baseline (speedup 1.0000x reference)
import functools

import jax
import jax.numpy as jnp
from jax import lax
from jax.experimental import pallas as pl
from jax.experimental.pallas import tpu as pltpu
from jax.experimental.pallas import tpu_sc as plsc

HIDDEN = 64
CHUNK = 128


def _gather_kernel(n_chunks_per_worker: int):
    mesh = plsc.VectorSubcoreMesh(core_axis_name="c", subcore_axis_name="s")
    n_rows = n_chunks_per_worker * 32 * CHUNK

    @functools.partial(
        pl.kernel,
        mesh=mesh,
        out_type=jax.ShapeDtypeStruct((n_rows, HIDDEN), jnp.float32),
        scratch_types=[
            pltpu.VMEM((n_chunks_per_worker, CHUNK), jnp.int32),
            pltpu.VMEM((CHUNK, HIDDEN), jnp.float32),
            pltpu.SemaphoreType.DMA,
        ],
        compiler_params=pltpu.CompilerParams(use_tc_tiling_on_sc=False),
    )
    def k(idx_hbm, table_hbm, out_hbm, idx_v, rows_v, sem):
        wid = lax.axis_index("s") * 2 + lax.axis_index("c")
        chunk_base = wid * n_chunks_per_worker
        pltpu.sync_copy(idx_hbm.at[pl.ds(chunk_base, n_chunks_per_worker)],
                        idx_v)

        def body(j, _):
            pltpu.async_copy(table_hbm.at[idx_v.at[j]], rows_v, sem).wait()
            pltpu.sync_copy(
                rows_v,
                out_hbm.at[pl.ds((chunk_base + j) * CHUNK, CHUNK)])
            return _

        lax.fori_loop(0, n_chunks_per_worker, body, 0)

    return k


def kernel(ids, table):
    b, s, _ = ids.shape
    n = b * s
    n_chunks = n // (32 * CHUNK)
    idx = ids.reshape(n // CHUNK, CHUNK).astype(jnp.int32)
    out = _gather_kernel(n_chunks)(idx, table)
    return out.reshape(b, s, HIDDEN)


# speedup vs baseline: 1.1142x; 1.1142x over previous
import functools

import jax
import jax.numpy as jnp
from jax import lax
from jax.experimental import pallas as pl
from jax.experimental.pallas import tpu as pltpu
from jax.experimental.pallas import tpu_sc as plsc

HIDDEN = 64
CHUNK = 128
K = 5
NW = 32


def _gather_kernel(n_chunks_per_worker: int):
    mesh = plsc.VectorSubcoreMesh(core_axis_name="c", subcore_axis_name="s")
    n_rows = n_chunks_per_worker * NW * CHUNK
    n_groups = n_chunks_per_worker // K
    assert n_groups % 2 == 0 and n_groups * K == n_chunks_per_worker

    @functools.partial(
        pl.kernel,
        mesh=mesh,
        out_type=jax.ShapeDtypeStruct((n_rows, HIDDEN), jnp.float32),
        scratch_types=[
            pltpu.VMEM((n_chunks_per_worker, CHUNK), jnp.int32),
            pltpu.VMEM((K * CHUNK, HIDDEN), jnp.float32),
            pltpu.VMEM((K * CHUNK, HIDDEN), jnp.float32),
            pltpu.SemaphoreType.DMA,
            pltpu.SemaphoreType.DMA,
            pltpu.SemaphoreType.DMA,
        ],
        compiler_params=pltpu.CompilerParams(use_tc_tiling_on_sc=False),
    )
    def k(idx_hbm, table_hbm, out_hbm, idx_v, bank0, bank1, sg0, sg1, ss):
        wid = lax.axis_index("s") * 2 + lax.axis_index("c")
        chunk_base = wid * n_chunks_per_worker
        banks = (bank0, bank1)
        gsems = (sg0, sg1)

        pltpu.sync_copy(idx_hbm.at[pl.ds(chunk_base, n_chunks_per_worker)],
                        idx_v)

        def fire_gathers(g, bank, sem):
            for j in range(K):
                pltpu.async_copy(
                    table_hbm.at[idx_v.at[g * K + j]],
                    bank.at[pl.ds(j * CHUNK, CHUNK)], sem)

        def drain_gathers(bank, sem):
            for j in range(K):
                pltpu.make_async_copy(
                    table_hbm.at[idx_v.at[0]],
                    bank.at[pl.ds(j * CHUNK, CHUNK)], sem).wait()

        def out_slice(g):
            return out_hbm.at[pl.ds((chunk_base + g * K) * CHUNK, K * CHUNK)]

        def drain_scatter(bank):
            pltpu.make_async_copy(bank, out_slice(0), ss).wait()

        fire_gathers(0, bank0, sg0)

        def body(i, carry):
            for p in range(2):
                g = 2 * i + p
                cur, nxt = banks[p], banks[1 - p]
                drain_gathers(cur, gsems[p])
                @pl.when(g >= 1)
                def _():
                    drain_scatter(nxt)

                @pl.when(g + 1 < n_groups)
                def _():
                    fire_gathers(g + 1, nxt, gsems[1 - p])

                pltpu.async_copy(cur, out_slice(g), ss)
            return carry

        lax.fori_loop(0, n_groups // 2, body, 0)
        drain_scatter(bank1)

    return k


def kernel(ids, table):
    b, s, _ = ids.shape
    n = b * s
    n_chunks = n // (NW * CHUNK)
    idx = ids.reshape(n // CHUNK, CHUNK).astype(jnp.int32)
    out = _gather_kernel(n_chunks)(idx, table)
    return out.reshape(b, s, HIDDEN)
